# Initial kernel scaffold; baseline (speedup 1.0000x reference)
import jax
import jax.numpy as jnp
from jax.experimental import pallas as pl


def kernel(edge_parameters, stitch_types, panel_indices, edge_indices, W_edge, b_edge, panel_tab, edge_tab, stitch_tab, ln_gamma, ln_beta):
    raise NotImplementedError("write your pallas kernel here")


# speedup vs baseline: 2.6446x; 2.6446x over previous
import functools

import jax
import jax.numpy as jnp
from jax import lax
from jax.experimental import pallas as pl
from jax.experimental.pallas import tpu as pltpu

HIDDEN = 768
EDGE_FEAT = 9
MAX_PANELS = 37
MAX_EDGES = 39
NUM_STITCH = 8
CAT = 96


def _body(ep_ref, pidx_ref, eidx_ref, sidx_ref, w_ref, b_ref, tab_ref,
          g_ref, beta_ref, out_ref):
    T = ep_ref.shape[0]
    ep = ep_ref[...]
    acc = jnp.dot(ep, w_ref[...], preferred_element_type=jnp.float32)
    acc = acc + b_ref[...]

    p = pidx_ref[0]
    e = eidx_ref[0] + MAX_PANELS
    s = sidx_ref[0] + (MAX_PANELS + MAX_EDGES)
    cols = lax.broadcasted_iota(jnp.int32, (T, CAT), 1)
    oh = ((cols == p).astype(jnp.float32)
          + (cols == e).astype(jnp.float32)
          + (cols == s).astype(jnp.float32))
    acc = acc + jnp.dot(oh, tab_ref[...], preferred_element_type=jnp.float32)

    mean = jnp.mean(acc, axis=1, keepdims=True)
    cen = acc - mean
    var = jnp.mean(cen * cen, axis=1, keepdims=True)
    inv = lax.rsqrt(var + 1e-5)
    out_ref[...] = cen * inv * g_ref[...] + beta_ref[...]


@functools.partial(jax.jit, static_argnames=("T",))
def _run(ep, pidx, eidx, sidx, W, b, tab, gamma, beta, T=624):
    N = ep.shape[0]
    grid = N // T
    pidx3 = pidx.reshape(grid, T, 1)
    eidx3 = eidx.reshape(grid, T, 1)
    sidx3 = sidx.reshape(grid, T, 1)
    tok_spec = pl.BlockSpec((T, EDGE_FEAT), lambda i: (i, 0))
    idx_spec = pl.BlockSpec((1, T, 1), lambda i: (i, 0, 0))
    full = lambda shape: pl.BlockSpec(shape, lambda i: (0,) * len(shape))
    return pl.pallas_call(
        _body,
        grid=(grid,),
        in_specs=[
            tok_spec, idx_spec, idx_spec, idx_spec,
            full((EDGE_FEAT, HIDDEN)),
            full((1, HIDDEN)),
            full((CAT, HIDDEN)),
            full((1, HIDDEN)),
            full((1, HIDDEN)),
        ],
        out_specs=pl.BlockSpec((T, HIDDEN), lambda i: (i, 0)),
        out_shape=jax.ShapeDtypeStruct((N, HIDDEN), jnp.float32),
    )(ep, pidx3, eidx3, sidx3, W, b, tab, gamma, beta)


def kernel(edge_parameters, stitch_types, panel_indices, edge_indices,
           W_edge, b_edge, panel_tab, edge_tab, stitch_tab, ln_gamma, ln_beta):
    B, P, E, F = edge_parameters.shape
    N = B * P * E
    ep = edge_parameters.reshape(N, F)
    pidx = panel_indices.reshape(N).astype(jnp.int32)
    eidx = edge_indices.reshape(N).astype(jnp.int32)
    sidx = stitch_types.reshape(N).astype(jnp.int32)
    tab = jnp.zeros((CAT, HIDDEN), jnp.float32)
    tab = tab.at[:MAX_PANELS].set(panel_tab)
    tab = tab.at[MAX_PANELS:MAX_PANELS + MAX_EDGES].set(edge_tab)
    tab = tab.at[MAX_PANELS + MAX_EDGES:MAX_PANELS + MAX_EDGES + NUM_STITCH].set(stitch_tab)
    out = _run(ep, pidx, eidx, sidx, W_edge, b_edge.reshape(1, HIDDEN), tab,
               ln_gamma.reshape(1, HIDDEN), ln_beta.reshape(1, HIDDEN))
    return out.reshape(B, P, E, HIDDEN).reshape(B, P * E, HIDDEN)


# speedup vs baseline: 4.5054x; 1.7036x over previous
import jax
import jax.numpy as jnp
from jax import lax
from jax.experimental import pallas as pl

HIDDEN = 768
EDGE_FEAT = 9
MAX_PANELS = 37
MAX_EDGES = 39
NUM_STITCH = 8
CAT = 96


def _body(ep_ref, pidx_ref, eidx_ref, sidx_ref, w_ref, b_ref, tab_ref,
          g_ref, beta_ref, out_ref):
    T = ep_ref.shape[1]
    ep = ep_ref[0]
    acc = jnp.dot(ep, w_ref[...], preferred_element_type=jnp.float32)
    acc = acc + b_ref[...]

    p = pidx_ref[0]
    e = eidx_ref[0] + MAX_PANELS
    s = sidx_ref[0] + (MAX_PANELS + MAX_EDGES)
    cols = lax.broadcasted_iota(jnp.int32, (T, CAT), 1)
    oh = ((cols == p).astype(jnp.float32)
          + (cols == e).astype(jnp.float32)
          + (cols == s).astype(jnp.float32))
    acc = acc + jnp.dot(oh, tab_ref[...], preferred_element_type=jnp.float32)

    mean = jnp.mean(acc, axis=1, keepdims=True)
    cen = acc - mean
    var = jnp.mean(cen * cen, axis=1, keepdims=True)
    inv = lax.rsqrt(var + 1e-5)
    out_ref[0] = cen * inv * g_ref[...] + beta_ref[...]


def kernel(edge_parameters, stitch_types, panel_indices, edge_indices,
           W_edge, b_edge, panel_tab, edge_tab, stitch_tab, ln_gamma, ln_beta):
    B, P, E, F = edge_parameters.shape
    T = P * E
    N = B * T
    ep = edge_parameters.reshape(B, T, F)
    pidx = panel_indices.reshape(B, T, 1).astype(jnp.int32)
    eidx = edge_indices.reshape(B, T, 1).astype(jnp.int32)
    sidx = stitch_types.reshape(B, T, 1).astype(jnp.int32)
    tab = jnp.concatenate(
        [panel_tab, edge_tab, stitch_tab,
         jnp.zeros((CAT - MAX_PANELS - MAX_EDGES - NUM_STITCH, HIDDEN),
                   jnp.float32)], axis=0)

    tok_spec = pl.BlockSpec((1, T, EDGE_FEAT), lambda i: (i, 0, 0))
    idx_spec = pl.BlockSpec((1, T, 1), lambda i: (i, 0, 0))
    full = lambda shape: pl.BlockSpec(shape, lambda i: (0,) * len(shape))
    out = pl.pallas_call(
        _body,
        grid=(B,),
        in_specs=[
            tok_spec, idx_spec, idx_spec, idx_spec,
            full((EDGE_FEAT, HIDDEN)),
            full((1, HIDDEN)),
            full((CAT, HIDDEN)),
            full((1, HIDDEN)),
            full((1, HIDDEN)),
        ],
        out_specs=pl.BlockSpec((1, T, HIDDEN), lambda i: (i, 0, 0)),
        out_shape=jax.ShapeDtypeStruct((B, T, HIDDEN), jnp.float32),
    )(ep, pidx, eidx, sidx, W_edge, b_edge.reshape(1, HIDDEN), tab,
      ln_gamma.reshape(1, HIDDEN), ln_beta.reshape(1, HIDDEN))
    return out


# speedup vs baseline: 4.6259x; 1.0267x over previous
import jax
import jax.numpy as jnp
from jax import lax
from jax.experimental import pallas as pl

HIDDEN = 768
EDGE_FEAT = 9
MAX_PANELS = 37
MAX_EDGES = 39
NUM_STITCH = 8
CAT = 96


def _body(ep_ref, pidx_ref, eidx_ref, sidx_ref, w_ref, b_ref, tab_ref,
          g_ref, beta_ref, out_ref):
    BB, T = ep_ref.shape[0], ep_ref.shape[1]
    for bb in range(BB):
        ep = ep_ref[bb]
        acc = jnp.dot(ep, w_ref[...], preferred_element_type=jnp.float32)
        acc = acc + b_ref[...]

        p = pidx_ref[bb]
        e = eidx_ref[bb] + MAX_PANELS
        s = sidx_ref[bb] + (MAX_PANELS + MAX_EDGES)
        cols = lax.broadcasted_iota(jnp.int32, (T, CAT), 1)
        oh = ((cols == p).astype(jnp.float32)
              + (cols == e).astype(jnp.float32)
              + (cols == s).astype(jnp.float32))
        acc = acc + jnp.dot(oh, tab_ref[...], preferred_element_type=jnp.float32)

        mean = jnp.mean(acc, axis=1, keepdims=True)
        cen = acc - mean
        var = jnp.mean(cen * cen, axis=1, keepdims=True)
        inv = lax.rsqrt(var + 1e-5)
        out_ref[bb] = cen * inv * g_ref[...] + beta_ref[...]


def kernel(edge_parameters, stitch_types, panel_indices, edge_indices,
           W_edge, b_edge, panel_tab, edge_tab, stitch_tab, ln_gamma, ln_beta):
    B, P, E, F = edge_parameters.shape
    T = P * E
    N = B * T
    ep = edge_parameters.reshape(B, T, F)
    pidx = panel_indices.reshape(B, T, 1).astype(jnp.int32)
    eidx = edge_indices.reshape(B, T, 1).astype(jnp.int32)
    sidx = stitch_types.reshape(B, T, 1).astype(jnp.int32)
    tab = jnp.concatenate(
        [panel_tab, edge_tab, stitch_tab,
         jnp.zeros((CAT - MAX_PANELS - MAX_EDGES - NUM_STITCH, HIDDEN),
                   jnp.float32)], axis=0)

    BB = 2
    tok_spec = pl.BlockSpec((BB, T, EDGE_FEAT), lambda i: (i, 0, 0))
    idx_spec = pl.BlockSpec((BB, T, 1), lambda i: (i, 0, 0))
    full = lambda shape: pl.BlockSpec(shape, lambda i: (0,) * len(shape))
    out = pl.pallas_call(
        _body,
        grid=(B // BB,),
        in_specs=[
            tok_spec, idx_spec, idx_spec, idx_spec,
            full((EDGE_FEAT, HIDDEN)),
            full((1, HIDDEN)),
            full((CAT, HIDDEN)),
            full((1, HIDDEN)),
            full((1, HIDDEN)),
        ],
        out_specs=pl.BlockSpec((BB, T, HIDDEN), lambda i: (i, 0, 0)),
        out_shape=jax.ShapeDtypeStruct((B, T, HIDDEN), jnp.float32),
    )(ep, pidx, eidx, sidx, W_edge, b_edge.reshape(1, HIDDEN), tab,
      ln_gamma.reshape(1, HIDDEN), ln_beta.reshape(1, HIDDEN))
    return out
